# baseline (speedup 1.0000x reference)
import functools

import jax
import jax.numpy as jnp
from jax import lax
from jax.experimental import pallas as pl
from jax.experimental.pallas import tpu as pltpu
from jax.experimental.pallas import tpu_sc as plsc

NUM_NODES = 50000
EMB = 16
N_EDGES = 3200000
BATCH = 16384

NC = 2
NS = 16
LANES = 16

EPT = N_EDGES // NS
CHUNK = 1000
NCHUNKS = EPT // CHUNK
NITER = NCHUNKS // 4
ROWS_PER_TILE = NUM_NODES // NS

BSUB = 128
BPT = BATCH // (NC * NS)
BK = BPT // BSUB

_mesh = plsc.VectorSubcoreMesh(core_axis_name="c", subcore_axis_name="s")
_cparams = pltpu.CompilerParams(use_tc_tiling_on_sc=False)


def _layer_body(tbl, gidx, sidx, w, out, acc,
                gb0, gb1, gb2, gb3, sb0, sb1, sb2, sb3,
                wb0, wb1, wb2, wb3, rb0, rb1, rb2, rb3,
                sem_lg0, sem_lg1, sem_ls0, sem_ls1,
                sem_g0, sem_g1, sem_s0, sem_s1):
    c = lax.axis_index("c")
    s = lax.axis_index("s")
    gbufs = (gb0, gb1, gb2, gb3)
    sbufs = (sb0, sb1, sb2, sb3)
    wbufs = (wb0, wb1, wb2, wb3)
    rbufs = (rb0, rb1, rb2, rb3)
    sem_lgs = (sem_lg0, sem_lg1)
    sem_lss = (sem_ls0, sem_ls1)
    sem_gs = (sem_g0, sem_g1)
    sem_ss = (sem_s0, sem_s1)
    zero = jnp.zeros((LANES,), jnp.float32)

    def ling_copies(k, slot, sem):
        e0 = s * EPT + k * CHUNK
        return (pltpu.make_async_copy(gidx.at[c, pl.ds(e0, CHUNK)],
                                      gbufs[slot], sem),
                pltpu.make_async_copy(w.at[pl.ds(e0, CHUNK)],
                                      wbufs[slot], sem))

    def lins_copy(k, slot, sem):
        e0 = s * EPT + k * CHUNK
        return pltpu.make_async_copy(sidx.at[c, pl.ds(e0, CHUNK)],
                                     sbufs[slot], sem)

    def gath_copy(slot, rslot, sem):
        return pltpu.make_async_copy(tbl.at[gbufs[slot]], rbufs[rslot], sem)

    def scat_copy(slot, rslot, sem):
        return pltpu.make_async_copy(rbufs[rslot], acc.at[sbufs[slot]], sem)

    def scat_fire(slot, rslot, sem):
        pltpu.async_copy(rbufs[rslot], acc.at[sbufs[slot]], sem, add=True)

    def mul(slot, rslot):
        wb, rb = wbufs[slot], rbufs[rslot]

        def body(g, carry):
            wv = wb[pl.ds(g * LANES, LANES)]
            base = g * LANES
            for l in range(LANES):
                rb[base + l] = rb[base + l] * wv[l]
            return carry

        lax.fori_loop(0, CHUNK // LANES, body, 0)
        rem = CHUNK % LANES
        if rem:
            wv = wb[pl.ds(CHUNK - LANES, LANES)]
            for l in range(LANES - rem, LANES):
                rb[CHUNK - LANES + l] = rb[CHUNK - LANES + l] * wv[l]

    def _zf(r, carry):
        rb0[r] = zero
        return carry

    lax.fori_loop(0, CHUNK, _zf, 0)
    row0 = s * ROWS_PER_TILE
    for z in range(ROWS_PER_TILE // CHUNK):
        pltpu.sync_copy(rb0.at[pl.ds(0, CHUNK)],
                        acc.at[pl.ds(row0 + z * CHUNK, CHUNK)])
    rem = ROWS_PER_TILE % CHUNK
    if rem:
        pltpu.sync_copy(rb0.at[pl.ds(0, rem)],
                        acc.at[pl.ds(row0 + ROWS_PER_TILE - rem, rem)])
    plsc.subcore_barrier()

    for cp in ling_copies(0, 0, sem_lg0):
        cp.start()
    for cp in ling_copies(0, 0, sem_lg0):
        cp.wait()
    gath_copy(0, 0, sem_g0).start()
    for cp in ling_copies(1, 1, sem_lg1):
        cp.start()
    for cp in ling_copies(1, 1, sem_lg1):
        cp.wait()
    gath_copy(1, 1, sem_g1).start()
    for cp in ling_copies(2, 2, sem_lg0):
        cp.start()
    lins_copy(0, 0, sem_ls0).start()
    lins_copy(1, 1, sem_ls1).start()

    def iter_body(t, carry):
        for i in range(4):
            k = 4 * t + i
            p = i % 2
            if i >= 2:
                scat_copy(i - 2, i - 2, sem_ss[p]).wait()
            else:
                @pl.when(t >= 1)
                def _(i=i, p=p):
                    scat_copy((i + 2) % 4, (i + 2) % 4, sem_ss[p]).wait()
            lins_copy(k, i, sem_lss[p]).wait()
            if i < 2:
                lins_copy(k + 2, i + 2, sem_lss[p]).start()
            else:
                @pl.when(t < NITER - 1)
                def _(i=i, p=p, k=k):
                    lins_copy(k + 2, (i + 2) % 4, sem_lss[p]).start()
            if i < 2:
                for cp in ling_copies(k + 2, i + 2, sem_lgs[p]):
                    cp.wait()
            else:
                @pl.when(t < NITER - 1)
                def _(i=i, p=p, k=k):
                    for cp in ling_copies(k + 2, (i + 2) % 4, sem_lgs[p]):
                        cp.wait()
            gath_copy(i, i, sem_gs[p]).wait()
            if i == 0:
                for cp in ling_copies(k + 3, i + 3, sem_lgs[1 - p]):
                    cp.start()
            else:
                @pl.when(t < NITER - 1)
                def _(i=i, p=p, k=k):
                    for cp in ling_copies(k + 3, (i + 3) % 4, sem_lgs[1 - p]):
                        cp.start()
            if i < 2:
                gath_copy(i + 2, i + 2, sem_gs[p]).start()
            else:
                @pl.when(t < NITER - 1)
                def _(i=i, p=p):
                    gath_copy((i + 2) % 4, (i + 2) % 4, sem_gs[p]).start()
            mul(i, i)
            scat_fire(i, i, sem_ss[p])
        return carry

    lax.fori_loop(0, NITER, iter_body, 0)
    scat_copy(2, 2, sem_s0).wait()
    scat_copy(3, 3, sem_s1).wait()
    plsc.subcore_barrier()

    dst0 = (1 - c) * NUM_NODES + row0
    pltpu.sync_copy(acc.at[pl.ds(row0, ROWS_PER_TILE)],
                    out.at[pl.ds(dst0, ROWS_PER_TILE)])


_layer = functools.partial(
    pl.kernel,
    out_type=jax.ShapeDtypeStruct((2 * NUM_NODES, EMB), jnp.float32),
    mesh=_mesh,
    scratch_types=[
        pltpu.VMEM_SHARED((NUM_NODES, EMB), jnp.float32),
    ] + [pltpu.VMEM((CHUNK,), jnp.int32)] * 8
      + [pltpu.VMEM((CHUNK,), jnp.float32)] * 4
      + [pltpu.VMEM((CHUNK, EMB), jnp.float32)] * 4
      + [pltpu.SemaphoreType.DMA] * 8,
    compiler_params=_cparams,
)(_layer_body)


def _epi_body(t0, t1, t2, t3, uix, iix, uf, fi, u_init, i_init,
              uix_v, iix_v, ub0, ub1, ub2, ub3, ib0, ib1, ib2, ib3,
              uf_v, fi_v, sem):
    c = lax.axis_index("c")
    s = lax.axis_index("s")
    wid = s * NC + c

    r0 = wid * BK
    pltpu.sync_copy(uix.at[pl.ds(r0, BK)], uix_v)
    pltpu.sync_copy(iix.at[pl.ds(r0, BK)], iix_v)
    cps = []
    for j in range(BK):
        for t, ub, ib in zip((t0, t1, t2, t3),
                             (ub0, ub1, ub2, ub3),
                             (ib0, ib1, ib2, ib3)):
            cps.append(pltpu.async_copy(t.at[uix_v.at[j]],
                                        ub.at[pl.ds(j * BSUB, BSUB)], sem))
            cps.append(pltpu.async_copy(t.at[iix_v.at[j]],
                                        ib.at[pl.ds(j * BSUB, BSUB)], sem))
    for cp in cps:
        cp.wait()

    def _row(e, carry):
        uf_v[e] = (ub0[e] + ub1[e] + ub2[e] + ub3[e]) * 0.25
        fi_v[e] = (ib0[e] + ib1[e] + ib2[e] + ib3[e]) * 0.25
        return carry

    lax.fori_loop(0, BPT, _row, 0)

    b0 = wid * BPT
    pltpu.sync_copy(uf_v, uf.at[pl.ds(b0, BPT)])
    pltpu.sync_copy(fi_v, fi.at[pl.ds(b0, BPT)])
    pltpu.sync_copy(ub0, u_init.at[pl.ds(b0, BPT)])
    pltpu.sync_copy(ib0, i_init.at[pl.ds(b0, BPT)])


_epilogue = functools.partial(
    pl.kernel,
    out_type=(
        jax.ShapeDtypeStruct((BATCH, EMB), jnp.float32),
        jax.ShapeDtypeStruct((BATCH, EMB), jnp.float32),
        jax.ShapeDtypeStruct((BATCH, EMB), jnp.float32),
        jax.ShapeDtypeStruct((BATCH, EMB), jnp.float32),
    ),
    mesh=_mesh,
    scratch_types=[
        pltpu.VMEM((BK, BSUB), jnp.int32),
        pltpu.VMEM((BK, BSUB), jnp.int32),
    ] + [pltpu.VMEM((BPT, EMB), jnp.float32)] * 10 + [
        pltpu.SemaphoreType.DMA,
    ],
    compiler_params=_cparams,
)(_epi_body)


def _scores_body(a_ref, b_ref, o_ref):
    o_ref[...] = jnp.sum(a_ref[...] * b_ref[...], axis=-1)


_scores = pl.pallas_call(
    _scores_body,
    out_shape=jax.ShapeDtypeStruct((BATCH,), jnp.float32),
)


def kernel(user_table, item_table, weights, user_indices, item_indices,
           edge_index):
    edge_index = edge_index.astype(jnp.int32)
    src = edge_index[0]
    dst = edge_index[1]
    gidx = jnp.stack([dst, src + NUM_NODES])
    sidx = edge_index
    w = weights

    t0 = jnp.concatenate([item_table, user_table], axis=0)
    t1 = _layer(t0, gidx, sidx, w)
    t2 = _layer(t1, gidx, sidx, w)
    t3 = _layer(t2, gidx, sidx, w)

    uix = (user_indices.astype(jnp.int32) + NUM_NODES).reshape(
        BATCH // BSUB, BSUB)
    iix = item_indices.astype(jnp.int32).reshape(BATCH // BSUB, BSUB)
    uf, fi, u_init, i_init = _epilogue(t0, t1, t2, t3, uix, iix)
    scores = _scores(uf, fi)
    return scores.reshape(BATCH), u_init, i_init


# speedup vs baseline: 1.0566x; 1.0566x over previous
import functools

import jax
import jax.numpy as jnp
from jax import lax
from jax.experimental import pallas as pl
from jax.experimental.pallas import tpu as pltpu
from jax.experimental.pallas import tpu_sc as plsc

NUM_NODES = 50000
EMB = 16
N_EDGES = 3200000
BATCH = 16384

NC = 2
NS = 16
LANES = 16

EPT = N_EDGES // NS
CHUNK = 1000
NCHUNKS = EPT // CHUNK
NITER = NCHUNKS // 4
ROWS_PER_TILE = NUM_NODES // NS

BSUB = 128
BPT = BATCH // (NC * NS)
BK = BPT // BSUB

_mesh = plsc.VectorSubcoreMesh(core_axis_name="c", subcore_axis_name="s")
_cparams = pltpu.CompilerParams(use_tc_tiling_on_sc=False)


def _layer_body(tbl, eidx, w, out, acc,
                gb0, gb1, gb2, gb3, sb0, sb1, sb2, sb3,
                wb0, wb1, wb2, wb3, rb0, rb1, rb2, rb3,
                sem_lg0, sem_lg1, sem_ls0, sem_ls1,
                sem_g0, sem_g1, sem_s0, sem_s1):
    c = lax.axis_index("c")
    s = lax.axis_index("s")
    gbufs = (gb0, gb1, gb2, gb3)
    sbufs = (sb0, sb1, sb2, sb3)
    wbufs = (wb0, wb1, wb2, wb3)
    rbufs = (rb0, rb1, rb2, rb3)
    sem_lgs = (sem_lg0, sem_lg1)
    sem_lss = (sem_ls0, sem_ls1)
    sem_gs = (sem_g0, sem_g1)
    sem_ss = (sem_s0, sem_s1)
    zero = jnp.zeros((LANES,), jnp.float32)

    def ling_copies(k, slot, sem):
        e0 = s * EPT + k * CHUNK
        return (pltpu.make_async_copy(eidx.at[1 - c, pl.ds(e0, CHUNK)],
                                      gbufs[slot], sem),
                pltpu.make_async_copy(w.at[pl.ds(e0, CHUNK)],
                                      wbufs[slot], sem))

    def lins_copy(k, slot, sem):
        e0 = s * EPT + k * CHUNK
        return pltpu.make_async_copy(eidx.at[c, pl.ds(e0, CHUNK)],
                                     sbufs[slot], sem)

    def gath_copy(slot, rslot, sem):
        return pltpu.make_async_copy(tbl.at[c].at[gbufs[slot]],
                                     rbufs[rslot], sem)

    def scat_copy(slot, rslot, sem):
        return pltpu.make_async_copy(rbufs[rslot], acc.at[sbufs[slot]], sem)

    def scat_fire(slot, rslot, sem):
        pltpu.async_copy(rbufs[rslot], acc.at[sbufs[slot]], sem, add=True)

    def mul(slot, rslot):
        wb, rb = wbufs[slot], rbufs[rslot]

        def body(g, carry):
            wv = wb[pl.ds(g * LANES, LANES)]
            base = g * LANES
            for l in range(LANES):
                rb[base + l] = rb[base + l] * wv[l]
            return carry

        lax.fori_loop(0, CHUNK // LANES, body, 0)
        rem = CHUNK % LANES
        if rem:
            wv = wb[pl.ds(CHUNK - LANES, LANES)]
            for l in range(LANES - rem, LANES):
                rb[CHUNK - LANES + l] = rb[CHUNK - LANES + l] * wv[l]

    def _zf(r, carry):
        rb0[r] = zero
        return carry

    lax.fori_loop(0, CHUNK, _zf, 0)
    row0 = s * ROWS_PER_TILE
    for z in range(ROWS_PER_TILE // CHUNK):
        pltpu.sync_copy(rb0.at[pl.ds(0, CHUNK)],
                        acc.at[pl.ds(row0 + z * CHUNK, CHUNK)])
    rem = ROWS_PER_TILE % CHUNK
    if rem:
        pltpu.sync_copy(rb0.at[pl.ds(0, rem)],
                        acc.at[pl.ds(row0 + ROWS_PER_TILE - rem, rem)])
    plsc.subcore_barrier()

    for cp in ling_copies(0, 0, sem_lg0):
        cp.start()
    for cp in ling_copies(0, 0, sem_lg0):
        cp.wait()
    gath_copy(0, 0, sem_g0).start()
    for cp in ling_copies(1, 1, sem_lg1):
        cp.start()
    for cp in ling_copies(1, 1, sem_lg1):
        cp.wait()
    gath_copy(1, 1, sem_g1).start()
    for cp in ling_copies(2, 2, sem_lg0):
        cp.start()
    lins_copy(0, 0, sem_ls0).start()
    lins_copy(1, 1, sem_ls1).start()

    def iter_body(t, carry):
        for i in range(4):
            k = 4 * t + i
            p = i % 2
            if i >= 2:
                scat_copy(i - 2, i - 2, sem_ss[p]).wait()
            else:
                @pl.when(t >= 1)
                def _(i=i, p=p):
                    scat_copy((i + 2) % 4, (i + 2) % 4, sem_ss[p]).wait()
            lins_copy(k, i, sem_lss[p]).wait()
            if i < 2:
                lins_copy(k + 2, i + 2, sem_lss[p]).start()
            else:
                @pl.when(t < NITER - 1)
                def _(i=i, p=p, k=k):
                    lins_copy(k + 2, (i + 2) % 4, sem_lss[p]).start()
            if i < 2:
                for cp in ling_copies(k + 2, i + 2, sem_lgs[p]):
                    cp.wait()
            else:
                @pl.when(t < NITER - 1)
                def _(i=i, p=p, k=k):
                    for cp in ling_copies(k + 2, (i + 2) % 4, sem_lgs[p]):
                        cp.wait()
            gath_copy(i, i, sem_gs[p]).wait()
            if i == 0:
                for cp in ling_copies(k + 3, i + 3, sem_lgs[1 - p]):
                    cp.start()
            else:
                @pl.when(t < NITER - 1)
                def _(i=i, p=p, k=k):
                    for cp in ling_copies(k + 3, (i + 3) % 4, sem_lgs[1 - p]):
                        cp.start()
            if i < 2:
                gath_copy(i + 2, i + 2, sem_gs[p]).start()
            else:
                @pl.when(t < NITER - 1)
                def _(i=i, p=p):
                    gath_copy((i + 2) % 4, (i + 2) % 4, sem_gs[p]).start()
            mul(i, i)
            scat_fire(i, i, sem_ss[p])
        return carry

    lax.fori_loop(0, NITER, iter_body, 0)
    scat_copy(2, 2, sem_s0).wait()
    scat_copy(3, 3, sem_s1).wait()
    plsc.subcore_barrier()

    pltpu.sync_copy(acc.at[pl.ds(row0, ROWS_PER_TILE)],
                    out.at[1 - c, pl.ds(row0, ROWS_PER_TILE)])


_layer = functools.partial(
    pl.kernel,
    out_type=jax.ShapeDtypeStruct((2, NUM_NODES, EMB), jnp.float32),
    mesh=_mesh,
    scratch_types=[
        pltpu.VMEM_SHARED((NUM_NODES, EMB), jnp.float32),
    ] + [pltpu.VMEM((CHUNK,), jnp.int32)] * 8
      + [pltpu.VMEM((CHUNK,), jnp.float32)] * 4
      + [pltpu.VMEM((CHUNK, EMB), jnp.float32)] * 4
      + [pltpu.SemaphoreType.DMA] * 8,
    compiler_params=_cparams,
)(_layer_body)


def _epi_body(t0, t1, t2, t3, uix, iix, uf, fi, u_init, i_init,
              uix_v, iix_v, ub0, ub1, ub2, ub3, ib0, ib1, ib2, ib3,
              uf_v, fi_v, sem):
    c = lax.axis_index("c")
    s = lax.axis_index("s")
    wid = s * NC + c

    r0 = wid * BK
    pltpu.sync_copy(uix.at[pl.ds(r0, BK)], uix_v)
    pltpu.sync_copy(iix.at[pl.ds(r0, BK)], iix_v)
    cps = []
    for j in range(BK):
        for t, ub, ib in zip((t0, t1, t2, t3),
                             (ub0, ub1, ub2, ub3),
                             (ib0, ib1, ib2, ib3)):
            cps.append(pltpu.async_copy(t.at[1].at[uix_v.at[j]],
                                        ub.at[pl.ds(j * BSUB, BSUB)], sem))
            cps.append(pltpu.async_copy(t.at[0].at[iix_v.at[j]],
                                        ib.at[pl.ds(j * BSUB, BSUB)], sem))
    for cp in cps:
        cp.wait()

    def _row(e, carry):
        uf_v[e] = (ub0[e] + ub1[e] + ub2[e] + ub3[e]) * 0.25
        fi_v[e] = (ib0[e] + ib1[e] + ib2[e] + ib3[e]) * 0.25
        return carry

    lax.fori_loop(0, BPT, _row, 0)

    b0 = wid * BPT
    pltpu.sync_copy(uf_v, uf.at[pl.ds(b0, BPT)])
    pltpu.sync_copy(fi_v, fi.at[pl.ds(b0, BPT)])
    pltpu.sync_copy(ub0, u_init.at[pl.ds(b0, BPT)])
    pltpu.sync_copy(ib0, i_init.at[pl.ds(b0, BPT)])


_epilogue = functools.partial(
    pl.kernel,
    out_type=(
        jax.ShapeDtypeStruct((BATCH, EMB), jnp.float32),
        jax.ShapeDtypeStruct((BATCH, EMB), jnp.float32),
        jax.ShapeDtypeStruct((BATCH, EMB), jnp.float32),
        jax.ShapeDtypeStruct((BATCH, EMB), jnp.float32),
    ),
    mesh=_mesh,
    scratch_types=[
        pltpu.VMEM((BK, BSUB), jnp.int32),
        pltpu.VMEM((BK, BSUB), jnp.int32),
    ] + [pltpu.VMEM((BPT, EMB), jnp.float32)] * 10 + [
        pltpu.SemaphoreType.DMA,
    ],
    compiler_params=_cparams,
)(_epi_body)


def _scores_body(a_ref, b_ref, o_ref):
    o_ref[...] = jnp.sum(a_ref[...] * b_ref[...], axis=-1)


_scores = pl.pallas_call(
    _scores_body,
    out_shape=jax.ShapeDtypeStruct((BATCH,), jnp.float32),
)


def kernel(user_table, item_table, weights, user_indices, item_indices,
           edge_index):
    eidx = edge_index.astype(jnp.int32)
    w = weights

    t0 = jnp.stack([item_table, user_table])
    t1 = _layer(t0, eidx, w)
    t2 = _layer(t1, eidx, w)
    t3 = _layer(t2, eidx, w)

    uix = user_indices.astype(jnp.int32).reshape(BATCH // BSUB, BSUB)
    iix = item_indices.astype(jnp.int32).reshape(BATCH // BSUB, BSUB)
    uf, fi, u_init, i_init = _epilogue(t0, t1, t2, t3, uix, iix)
    scores = _scores(uf, fi)
    return scores.reshape(BATCH), u_init, i_init
